# Initial kernel scaffold; baseline (speedup 1.0000x reference)
import jax
import jax.numpy as jnp
from jax.experimental import pallas as pl


def kernel(x, adj, W1, b1, W2, b2, Wa1, Wa2, n_communities):
    raise NotImplementedError("write your pallas kernel here")


# speedup vs baseline: 1.0147x; 1.0147x over previous
import functools

import jax
import jax.numpy as jnp
from jax import lax
from jax.experimental import pallas as pl
from jax.experimental.pallas import tpu as pltpu

_NC = 100


def _prep_kernel(xt_ref, xc_ref, W_ref, b_ref, Wa_ref,
                 st_ref, sc_ref, ct_ref, cc_ref):
    xt = xt_ref[...]
    xc = xc_ref[...]
    W = W_ref[...]
    b = b_ref[...]
    Wa = Wa_ref[...]
    st = jnp.dot(xt, W, preferred_element_type=jnp.float32)
    sc = jnp.dot(xc, W, preferred_element_type=jnp.float32)
    hx = jnp.tanh(jnp.dot(xt, Wa, preferred_element_type=jnp.float32))
    hc = jnp.tanh(jnp.dot(xc, Wa, preferred_element_type=jnp.float32))
    scores = lax.dot_general(hx, hc, (((1,), (1,)), ((), ())),
                             preferred_element_type=jnp.float32)
    m = jnp.max(scores, axis=-1, keepdims=True)
    e = jnp.exp(scores - m)
    a = e / jnp.sum(e, axis=-1, keepdims=True)
    ct_ref[...] = jnp.dot(a, sc, preferred_element_type=jnp.float32) + b
    cc_ref[...] = lax.dot_general(a, st, (((0,), (0,)), ((), ())),
                                  preferred_element_type=jnp.float32) + b
    st_ref[...] = st.astype(jnp.bfloat16)
    sc_ref[...] = sc.astype(jnp.bfloat16)


def _prep(xt, xc, W, b, Wa):
    nr = xt.shape[0]
    nc = xc.shape[0]
    D = W.shape[1]
    return pl.pallas_call(
        _prep_kernel,
        out_shape=[
            jax.ShapeDtypeStruct((nr, D), jnp.bfloat16),
            jax.ShapeDtypeStruct((nc, D), jnp.bfloat16),
            jax.ShapeDtypeStruct((nr, D), jnp.float32),
            jax.ShapeDtypeStruct((nc, D), jnp.float32),
        ],
    )(xt, xc, W, b.reshape(1, D), Wa)


def _agg_kernel(adj_ref, s_ref, corr_ref, o_ref, *, relu):
    acc = jnp.dot(adj_ref[...].astype(jnp.bfloat16), s_ref[...],
                  preferred_element_type=jnp.float32) + corr_ref[...]
    o_ref[...] = jnp.maximum(acc, 0.0) if relu else acc


def _aggregate(adj, s_bf, corr, relu, bm=400):
    n = adj.shape[0]
    D = s_bf.shape[1]
    return pl.pallas_call(
        functools.partial(_agg_kernel, relu=relu),
        grid=(n // bm,),
        in_specs=[
            pl.BlockSpec((bm, n), lambda i: (i, 0)),
            pl.BlockSpec((n, D), lambda i: (0, 0)),
            pl.BlockSpec((bm, D), lambda i: (i, 0)),
        ],
        out_specs=pl.BlockSpec((bm, D), lambda i: (i, 0)),
        out_shape=jax.ShapeDtypeStruct((n, D), jnp.float32),
        compiler_params=pltpu.CompilerParams(
            dimension_semantics=("parallel",),
        ),
    )(adj, s_bf, corr)


def kernel(x, adj, W1, b1, W2, b2, Wa1, Wa2, n_communities):
    nc = _NC
    xt, xc = x[:-nc], x[-nc:]
    s1t, s1c, c1t, c1c = _prep(xt, xc, W1, b1, Wa1)
    s1 = jnp.concatenate([s1t, s1c], axis=0)
    c1 = jnp.concatenate([c1t, c1c], axis=0)
    h = _aggregate(adj, s1, c1, relu=True)

    shift = jnp.asarray(n_communities, jnp.float32) - jnp.float32(nc)
    ht, hcm = h[:-nc], h[-nc:]
    s2t, s2c, c2t, c2c = _prep(ht, hcm, W2, b2 + shift, Wa2)
    s2 = jnp.concatenate([s2t, s2c], axis=0)
    c2 = jnp.concatenate([c2t, c2c], axis=0)
    out = _aggregate(adj, s2, c2, relu=False)
    return out


# speedup vs baseline: 1.1284x; 1.1121x over previous
import functools

import jax
import jax.numpy as jnp
from jax import lax
from jax.experimental import pallas as pl
from jax.experimental.pallas import tpu as pltpu

_NC = 100


def _prep_kernel(x_ref, W_ref, b_ref, Wa_ref, s_ref, c_ref):
    x = x_ref[...]
    W = W_ref[...]
    b = b_ref[...]
    Wa = Wa_ref[...]
    n = x.shape[0]
    nr = n - _NC

    s = jnp.dot(x, W, preferred_element_type=jnp.float32)
    z = jnp.tanh(jnp.dot(x, Wa, preferred_element_type=jnp.float32))
    zc = lax.slice(z, (nr, 0), (n, z.shape[1]))
    scores = lax.dot_general(z, zc, (((1,), (1,)), ((), ())),
                             preferred_element_type=jnp.float32)
    m = jnp.max(scores, axis=-1, keepdims=True)
    e = jnp.exp(scores - m)
    a = e / jnp.sum(e, axis=-1, keepdims=True)
    row = lax.broadcasted_iota(jnp.int32, (n, 1), 0)
    a = jnp.where(row < nr, a, 0.0)

    sc = lax.slice(s, (nr, 0), (n, s.shape[1]))
    c_ref[...] = jnp.dot(a, sc, preferred_element_type=jnp.float32) + b
    cc = lax.dot_general(a, s, (((0,), (0,)), ((), ())),
                         preferred_element_type=jnp.float32)
    c_ref[pl.ds(nr, _NC), :] = cc + b
    s_ref[...] = s.astype(jnp.bfloat16)


def _prep(x, W, b, Wa):
    n = x.shape[0]
    D = W.shape[1]
    return pl.pallas_call(
        _prep_kernel,
        out_shape=[
            jax.ShapeDtypeStruct((n, D), jnp.bfloat16),
            jax.ShapeDtypeStruct((n, D), jnp.float32),
        ],
    )(x, W, b.reshape(1, D), Wa)


def _agg_kernel(adj_ref, s_ref, corr_ref, o_ref, *, relu):
    acc = jnp.dot(adj_ref[...].astype(jnp.bfloat16), s_ref[...],
                  preferred_element_type=jnp.float32) + corr_ref[...]
    o_ref[...] = jnp.maximum(acc, 0.0) if relu else acc


def _aggregate(adj, s_bf, corr, relu, bm=400):
    n = adj.shape[0]
    D = s_bf.shape[1]
    return pl.pallas_call(
        functools.partial(_agg_kernel, relu=relu),
        grid=(n // bm,),
        in_specs=[
            pl.BlockSpec((bm, n), lambda i: (i, 0)),
            pl.BlockSpec((n, D), lambda i: (0, 0)),
            pl.BlockSpec((bm, D), lambda i: (i, 0)),
        ],
        out_specs=pl.BlockSpec((bm, D), lambda i: (i, 0)),
        out_shape=jax.ShapeDtypeStruct((n, D), jnp.float32),
        compiler_params=pltpu.CompilerParams(
            dimension_semantics=("parallel",),
        ),
    )(adj, s_bf, corr)


def kernel(x, adj, W1, b1, W2, b2, Wa1, Wa2, n_communities):
    s1, c1 = _prep(x, W1, b1, Wa1)
    h = _aggregate(adj, s1, c1, relu=True)

    shift = jnp.asarray(n_communities, jnp.float32) - jnp.float32(_NC)
    s2, c2 = _prep(h, W2, b2 + shift, Wa2)
    out = _aggregate(adj, s2, c2, relu=False)
    return out


# speedup vs baseline: 1.1899x; 1.0545x over previous
import functools

import jax
import jax.numpy as jnp
from jax import lax
from jax.experimental import pallas as pl
from jax.experimental.pallas import tpu as pltpu

_NC = 100


def _layer_kernel(x_ref, W_ref, b_ref, Wa_ref, adj_ref, o_ref, s_scr, c_scr,
                  *, bm, relu):
    i = pl.program_id(0)

    @pl.when(i == 0)
    def _prep():
        x = x_ref[...]
        W = W_ref[...]
        b = b_ref[...]
        Wa = Wa_ref[...]
        n = x.shape[0]
        nr = n - _NC

        s = jnp.dot(x, W, preferred_element_type=jnp.float32)
        z = jnp.tanh(jnp.dot(x, Wa, preferred_element_type=jnp.float32))
        zc = lax.slice(z, (nr, 0), (n, z.shape[1]))
        scores = lax.dot_general(z, zc, (((1,), (1,)), ((), ())),
                                 preferred_element_type=jnp.float32)
        m = jnp.max(scores, axis=-1, keepdims=True)
        e = jnp.exp(scores - m)
        a = e / jnp.sum(e, axis=-1, keepdims=True)
        row = lax.broadcasted_iota(jnp.int32, (n, 1), 0)
        a = jnp.where(row < nr, a, 0.0)

        sc = lax.slice(s, (nr, 0), (n, s.shape[1]))
        c_scr[...] = jnp.dot(a, sc, preferred_element_type=jnp.float32) + b
        cc = lax.dot_general(a, s, (((0,), (0,)), ((), ())),
                             preferred_element_type=jnp.float32)
        c_scr[pl.ds(nr, _NC), :] = cc + b
        s_scr[...] = s.astype(jnp.bfloat16)

    @pl.when(i > 0)
    def _agg():
        blk = i - 1
        acc = jnp.dot(adj_ref[...].astype(jnp.bfloat16), s_scr[...],
                      preferred_element_type=jnp.float32)
        acc = acc + c_scr[pl.ds(blk * bm, bm), :]
        o_ref[...] = jnp.maximum(acc, 0.0) if relu else acc


def _layer(x, W, b, Wa, adj, relu, bm=200):
    n = adj.shape[0]
    D = W.shape[1]
    zero = lambda i: (0, 0)
    prev = lambda i: (jnp.maximum(i - 1, 0), 0)
    return pl.pallas_call(
        functools.partial(_layer_kernel, bm=bm, relu=relu),
        grid=(n // bm + 1,),
        in_specs=[
            pl.BlockSpec((n, x.shape[1]), zero),
            pl.BlockSpec((x.shape[1], D), zero),
            pl.BlockSpec((1, D), zero),
            pl.BlockSpec((x.shape[1], Wa.shape[1]), zero),
            pl.BlockSpec((bm, n), prev),
        ],
        out_specs=pl.BlockSpec((bm, D), prev),
        out_shape=jax.ShapeDtypeStruct((n, D), jnp.float32),
        scratch_shapes=[
            pltpu.VMEM((n, D), jnp.bfloat16),
            pltpu.VMEM((n, D), jnp.float32),
        ],
        compiler_params=pltpu.CompilerParams(
            dimension_semantics=("arbitrary",),
        ),
    )(x, W, b.reshape(1, D), Wa, adj)


def kernel(x, adj, W1, b1, W2, b2, Wa1, Wa2, n_communities):
    h = _layer(x, W1, b1, Wa1, adj, relu=True)
    shift = jnp.asarray(n_communities, jnp.float32) - jnp.float32(_NC)
    out = _layer(h, W2, b2 + shift, Wa2, adj, relu=False)
    return out


# speedup vs baseline: 1.1922x; 1.0019x over previous
import functools

import jax
import jax.numpy as jnp
from jax import lax
from jax.experimental import pallas as pl
from jax.experimental.pallas import tpu as pltpu

_NC = 100


def _layer_kernel(x_ref, W_ref, b_ref, Wa_ref, adj_ref, o_ref, s_scr, c_scr,
                  *, bm, relu):
    i = pl.program_id(0)

    @pl.when(i == 0)
    def _prep():
        x = x_ref[...]
        W = W_ref[...]
        b = b_ref[...]
        Wa = Wa_ref[...]
        n = x.shape[0]
        nr = n - _NC

        s = jnp.dot(x, W, preferred_element_type=jnp.float32)
        z = jnp.tanh(jnp.dot(x, Wa, preferred_element_type=jnp.float32))
        zc = lax.slice(z, (nr, 0), (n, z.shape[1]))
        scores = lax.dot_general(z, zc, (((1,), (1,)), ((), ())),
                                 preferred_element_type=jnp.float32)
        m = jnp.max(scores, axis=-1, keepdims=True)
        e = jnp.exp(scores - m)
        a = e / jnp.sum(e, axis=-1, keepdims=True)
        row = lax.broadcasted_iota(jnp.int32, (n, 1), 0)
        a = jnp.where(row < nr, a, 0.0)

        sc = lax.slice(s, (nr, 0), (n, s.shape[1]))
        c_scr[...] = jnp.dot(a, sc, preferred_element_type=jnp.float32) + b
        cc = lax.dot_general(a, s, (((0,), (0,)), ((), ())),
                             preferred_element_type=jnp.float32)
        c_scr[pl.ds(nr, _NC), :] = cc + b
        s_scr[...] = s.astype(jnp.bfloat16)

    @pl.when(i > 0)
    def _agg():
        blk = i - 1
        acc = jnp.dot(adj_ref[...].astype(jnp.bfloat16), s_scr[...],
                      preferred_element_type=jnp.float32)
        acc = acc + c_scr[pl.ds(blk * bm, bm), :]
        o_ref[...] = jnp.maximum(acc, 0.0) if relu else acc


def _layer(x, W, b, Wa, adj, relu, bm=400):
    n = adj.shape[0]
    D = W.shape[1]
    zero = lambda i: (0, 0)
    prev = lambda i: (jnp.maximum(i - 1, 0), 0)
    return pl.pallas_call(
        functools.partial(_layer_kernel, bm=bm, relu=relu),
        grid=(n // bm + 1,),
        in_specs=[
            pl.BlockSpec((n, x.shape[1]), zero),
            pl.BlockSpec((x.shape[1], D), zero),
            pl.BlockSpec((1, D), zero),
            pl.BlockSpec((x.shape[1], Wa.shape[1]), zero),
            pl.BlockSpec((bm, n), prev),
        ],
        out_specs=pl.BlockSpec((bm, D), prev),
        out_shape=jax.ShapeDtypeStruct((n, D), jnp.float32),
        scratch_shapes=[
            pltpu.VMEM((n, D), jnp.bfloat16),
            pltpu.VMEM((n, D), jnp.float32),
        ],
        compiler_params=pltpu.CompilerParams(
            dimension_semantics=("arbitrary",),
        ),
    )(x, W, b.reshape(1, D), Wa, adj)


def kernel(x, adj, W1, b1, W2, b2, Wa1, Wa2, n_communities):
    h = _layer(x, W1, b1, Wa1, adj, relu=True)
    shift = jnp.asarray(n_communities, jnp.float32) - jnp.float32(_NC)
    out = _layer(h, W2, b2 + shift, Wa2, adj, relu=False)
    return out


# speedup vs baseline: 1.2110x; 1.0158x over previous
import functools

import jax
import jax.numpy as jnp
from jax import lax
from jax.experimental import pallas as pl
from jax.experimental.pallas import tpu as pltpu

_NC = 100


def _prep(x, W, b, Wa, s_scr, c_scr):
    n = x.shape[0]
    nr = n - _NC
    s = jnp.dot(x, W, preferred_element_type=jnp.float32)
    z = jnp.tanh(jnp.dot(x, Wa, preferred_element_type=jnp.float32))
    zc = lax.slice(z, (nr, 0), (n, z.shape[1]))
    scores = lax.dot_general(z, zc, (((1,), (1,)), ((), ())),
                             preferred_element_type=jnp.float32)
    m = jnp.max(scores, axis=-1, keepdims=True)
    e = jnp.exp(scores - m)
    a = e / jnp.sum(e, axis=-1, keepdims=True)
    row = lax.broadcasted_iota(jnp.int32, (n, 1), 0)
    a = jnp.where(row < nr, a, 0.0)

    sc = lax.slice(s, (nr, 0), (n, s.shape[1]))
    c_scr[...] = jnp.dot(a, sc, preferred_element_type=jnp.float32) + b
    cc = lax.dot_general(a, s, (((0,), (0,)), ((), ())),
                         preferred_element_type=jnp.float32)
    c_scr[pl.ds(nr, _NC), :] = cc + b
    s_scr[...] = s.astype(jnp.bfloat16)


def _gcn_kernel(x_ref, W1_ref, b1_ref, Wa1_ref, W2_ref, b2_ref, Wa2_ref,
                adj_ref, o_ref, s1_scr, c1_scr, h_scr, s2_scr, c2_scr,
                *, bm, nblk):
    i = pl.program_id(0)

    @pl.when(i == 0)
    def _prep1():
        _prep(x_ref[...], W1_ref[...], b1_ref[...], Wa1_ref[...],
              s1_scr, c1_scr)

    @pl.when((i >= 1) & (i <= nblk))
    def _agg1():
        blk = i - 1
        acc = jnp.dot(adj_ref[...].astype(jnp.bfloat16), s1_scr[...],
                      preferred_element_type=jnp.float32)
        acc = acc + c1_scr[pl.ds(blk * bm, bm), :]
        h_scr[pl.ds(blk * bm, bm), :] = jnp.maximum(acc, 0.0)

    @pl.when(i == nblk + 1)
    def _prep2():
        _prep(h_scr[...], W2_ref[...], b2_ref[...], Wa2_ref[...],
              s2_scr, c2_scr)

    @pl.when(i >= nblk + 2)
    def _agg2():
        blk = i - (nblk + 2)
        acc = jnp.dot(adj_ref[...].astype(jnp.bfloat16), s2_scr[...],
                      preferred_element_type=jnp.float32)
        o_ref[...] = acc + c2_scr[pl.ds(blk * bm, bm), :]


def kernel(x, adj, W1, b1, W2, b2, Wa1, Wa2, n_communities):
    n, d0 = x.shape
    d1 = W1.shape[1]
    d2 = W2.shape[1]
    bm = 200
    nblk = n // bm

    shift = jnp.asarray(n_communities, jnp.float32) - jnp.float32(_NC)
    b2_eff = (b2 + shift).reshape(1, d2)

    zero = lambda i: (0, 0)
    adj_idx = lambda i: (jnp.where(i <= nblk, jnp.maximum(i - 1, 0),
                                   jnp.maximum(i - (nblk + 2), 0)), 0)
    out_idx = lambda i: (jnp.maximum(i - (nblk + 2), 0), 0)

    return pl.pallas_call(
        functools.partial(_gcn_kernel, bm=bm, nblk=nblk),
        grid=(2 * nblk + 2,),
        in_specs=[
            pl.BlockSpec((n, d0), zero),
            pl.BlockSpec((d0, d1), zero),
            pl.BlockSpec((1, d1), zero),
            pl.BlockSpec((d0, Wa1.shape[1]), zero),
            pl.BlockSpec((d1, d2), zero),
            pl.BlockSpec((1, d2), zero),
            pl.BlockSpec((d1, Wa2.shape[1]), zero),
            pl.BlockSpec((bm, n), adj_idx),
        ],
        out_specs=pl.BlockSpec((bm, d2), out_idx),
        out_shape=jax.ShapeDtypeStruct((n, d2), jnp.float32),
        scratch_shapes=[
            pltpu.VMEM((n, d1), jnp.bfloat16),
            pltpu.VMEM((n, d1), jnp.float32),
            pltpu.VMEM((n, d1), jnp.float32),
            pltpu.VMEM((n, d2), jnp.bfloat16),
            pltpu.VMEM((n, d2), jnp.float32),
        ],
        compiler_params=pltpu.CompilerParams(
            dimension_semantics=("arbitrary",),
        ),
    )(x, W1, b1.reshape(1, d1), Wa1, W2, b2_eff, Wa2, adj)


# speedup vs baseline: 1.2738x; 1.0519x over previous
import functools

import jax
import jax.numpy as jnp
from jax import lax
from jax.experimental import pallas as pl
from jax.experimental.pallas import tpu as pltpu

_NC = 100
_NBUF = 4


def _prep(x, W, b, Wa, s_scr, c_scr):
    n = x.shape[0]
    nr = n - _NC
    s = jnp.dot(x, W, preferred_element_type=jnp.float32)
    z = jnp.tanh(jnp.dot(x, Wa, preferred_element_type=jnp.float32))
    zc = lax.slice(z, (nr, 0), (n, z.shape[1]))
    scores = lax.dot_general(z, zc, (((1,), (1,)), ((), ())),
                             preferred_element_type=jnp.float32)
    m = jnp.max(scores, axis=-1, keepdims=True)
    e = jnp.exp(scores - m)
    a = e / jnp.sum(e, axis=-1, keepdims=True)
    row = lax.broadcasted_iota(jnp.int32, (n, 1), 0)
    a = jnp.where(row < nr, a, 0.0)

    sc = lax.slice(s, (nr, 0), (n, s.shape[1]))
    c_scr[...] = jnp.dot(a, sc, preferred_element_type=jnp.float32) + b
    cc = lax.dot_general(a, s, (((0,), (0,)), ((), ())),
                         preferred_element_type=jnp.float32)
    c_scr[pl.ds(nr, _NC), :] = cc + b
    s_scr[...] = s.astype(jnp.bfloat16)


def _gcn_kernel(x_ref, W1_ref, b1_ref, Wa1_ref, W2_ref, b2_ref, Wa2_ref,
                adj_ref, o_ref, s1_scr, c1_scr, h_scr, s2_scr, c2_scr,
                abuf, sem, *, bm, nblk):
    i = pl.program_id(0)

    def fetch(t):
        blk = lax.rem(t, nblk)
        slot = lax.rem(t, _NBUF)
        pltpu.make_async_copy(
            adj_ref.at[pl.ds(blk * bm, bm), :], abuf.at[slot], sem.at[slot],
        ).start()

    def wait(t):
        blk = lax.rem(t, nblk)
        slot = lax.rem(t, _NBUF)
        pltpu.make_async_copy(
            adj_ref.at[pl.ds(blk * bm, bm), :], abuf.at[slot], sem.at[slot],
        ).wait()
        return slot

    @pl.when(i == 0)
    def _prep1():
        for t in range(_NBUF - 1):
            fetch(t)
        _prep(x_ref[...], W1_ref[...], b1_ref[...], Wa1_ref[...],
              s1_scr, c1_scr)

    def agg_step(t, s_scr, c_scr, store):
        nxt = t + (_NBUF - 1)

        @pl.when(nxt < 2 * nblk)
        def _():
            fetch(nxt)

        slot = wait(t)
        acc = jnp.dot(abuf[slot].astype(jnp.bfloat16), s_scr[...],
                      preferred_element_type=jnp.float32)
        store(lax.rem(t, nblk), acc + c_scr[pl.ds(lax.rem(t, nblk) * bm, bm), :])

    @pl.when((i >= 1) & (i <= nblk))
    def _agg1():
        def store(blk, v):
            h_scr[pl.ds(blk * bm, bm), :] = jnp.maximum(v, 0.0)
        agg_step(i - 1, s1_scr, c1_scr, store)

    @pl.when(i == nblk + 1)
    def _prep2():
        _prep(h_scr[...], W2_ref[...], b2_ref[...], Wa2_ref[...],
              s2_scr, c2_scr)

    @pl.when(i >= nblk + 2)
    def _agg2():
        def store(blk, v):
            o_ref[...] = v
        agg_step(i - 2, s2_scr, c2_scr, store)


def kernel(x, adj, W1, b1, W2, b2, Wa1, Wa2, n_communities):
    n, d0 = x.shape
    d1 = W1.shape[1]
    d2 = W2.shape[1]
    bm = 80
    nblk = n // bm

    shift = jnp.asarray(n_communities, jnp.float32) - jnp.float32(_NC)
    b2_eff = (b2 + shift).reshape(1, d2)

    zero = lambda i: (0, 0)
    out_idx = lambda i: (jnp.maximum(i - (nblk + 2), 0), 0)

    return pl.pallas_call(
        functools.partial(_gcn_kernel, bm=bm, nblk=nblk),
        grid=(2 * nblk + 2,),
        in_specs=[
            pl.BlockSpec((n, d0), zero),
            pl.BlockSpec((d0, d1), zero),
            pl.BlockSpec((1, d1), zero),
            pl.BlockSpec((d0, Wa1.shape[1]), zero),
            pl.BlockSpec((d1, d2), zero),
            pl.BlockSpec((1, d2), zero),
            pl.BlockSpec((d1, Wa2.shape[1]), zero),
            pl.BlockSpec(memory_space=pl.ANY),
        ],
        out_specs=pl.BlockSpec((bm, d2), out_idx),
        out_shape=jax.ShapeDtypeStruct((n, d2), jnp.float32),
        scratch_shapes=[
            pltpu.VMEM((n, d1), jnp.bfloat16),
            pltpu.VMEM((n, d1), jnp.float32),
            pltpu.VMEM((n, d1), jnp.float32),
            pltpu.VMEM((n, d2), jnp.bfloat16),
            pltpu.VMEM((n, d2), jnp.float32),
            pltpu.VMEM((_NBUF, bm, n), jnp.float32),
            pltpu.SemaphoreType.DMA((_NBUF,)),
        ],
        compiler_params=pltpu.CompilerParams(
            dimension_semantics=("arbitrary",),
        ),
    )(x, W1, b1.reshape(1, d1), Wa1, W2, b2_eff, Wa2, adj)


# speedup vs baseline: 1.2740x; 1.0001x over previous
import functools

import jax
import jax.numpy as jnp
from jax import lax
from jax.experimental import pallas as pl
from jax.experimental.pallas import tpu as pltpu

_NC = 100
_NBUF = 7


def _prep(x, W, b, Wa, s_scr, c_scr):
    n = x.shape[0]
    nr = n - _NC
    s = jnp.dot(x, W, preferred_element_type=jnp.float32)
    z = jnp.tanh(jnp.dot(x, Wa, preferred_element_type=jnp.float32))
    zc = lax.slice(z, (nr, 0), (n, z.shape[1]))
    scores = lax.dot_general(z, zc, (((1,), (1,)), ((), ())),
                             preferred_element_type=jnp.float32)
    m = jnp.max(scores, axis=-1, keepdims=True)
    e = jnp.exp(scores - m)
    a = e / jnp.sum(e, axis=-1, keepdims=True)
    row = lax.broadcasted_iota(jnp.int32, (n, 1), 0)
    a = jnp.where(row < nr, a, 0.0)

    sc = lax.slice(s, (nr, 0), (n, s.shape[1]))
    c_scr[...] = jnp.dot(a, sc, preferred_element_type=jnp.float32) + b
    cc = lax.dot_general(a, s, (((0,), (0,)), ((), ())),
                         preferred_element_type=jnp.float32)
    c_scr[pl.ds(nr, _NC), :] = cc + b
    s_scr[...] = s.astype(jnp.bfloat16)


def _gcn_kernel(x_ref, W1_ref, b1_ref, Wa1_ref, W2_ref, b2_ref, Wa2_ref,
                adj_ref, o_ref, s1_scr, c1_scr, h_scr, s2_scr, c2_scr,
                abuf, sem, *, bm, nblk):
    i = pl.program_id(0)

    def fetch(t):
        blk = lax.rem(t, nblk)
        slot = lax.rem(t, _NBUF)
        pltpu.make_async_copy(
            adj_ref.at[pl.ds(blk * bm, bm), :], abuf.at[slot], sem.at[slot],
        ).start()

    def wait(t):
        blk = lax.rem(t, nblk)
        slot = lax.rem(t, _NBUF)
        pltpu.make_async_copy(
            adj_ref.at[pl.ds(blk * bm, bm), :], abuf.at[slot], sem.at[slot],
        ).wait()
        return slot

    @pl.when(i == 0)
    def _prep1():
        for t in range(_NBUF - 1):
            fetch(t)
        _prep(x_ref[...], W1_ref[...], b1_ref[...], Wa1_ref[...],
              s1_scr, c1_scr)

    def agg_step(t, s_scr, c_scr, store):
        nxt = t + (_NBUF - 1)

        @pl.when(nxt < 2 * nblk)
        def _():
            fetch(nxt)

        slot = wait(t)
        acc = jnp.dot(abuf[slot].astype(jnp.bfloat16), s_scr[...],
                      preferred_element_type=jnp.float32)
        store(lax.rem(t, nblk), acc + c_scr[pl.ds(lax.rem(t, nblk) * bm, bm), :])

    @pl.when((i >= 1) & (i <= nblk))
    def _agg1():
        def store(blk, v):
            h_scr[pl.ds(blk * bm, bm), :] = jnp.maximum(v, 0.0)
        agg_step(i - 1, s1_scr, c1_scr, store)

    @pl.when(i == nblk + 1)
    def _prep2():
        _prep(h_scr[...], W2_ref[...], b2_ref[...], Wa2_ref[...],
              s2_scr, c2_scr)

    @pl.when(i >= nblk + 2)
    def _agg2():
        def store(blk, v):
            o_ref[...] = v
        agg_step(i - 2, s2_scr, c2_scr, store)


def kernel(x, adj, W1, b1, W2, b2, Wa1, Wa2, n_communities):
    n, d0 = x.shape
    d1 = W1.shape[1]
    d2 = W2.shape[1]
    bm = 80
    nblk = n // bm

    shift = jnp.asarray(n_communities, jnp.float32) - jnp.float32(_NC)
    b2_eff = (b2 + shift).reshape(1, d2)

    zero = lambda i: (0, 0)
    out_idx = lambda i: (jnp.maximum(i - (nblk + 2), 0), 0)

    return pl.pallas_call(
        functools.partial(_gcn_kernel, bm=bm, nblk=nblk),
        grid=(2 * nblk + 2,),
        in_specs=[
            pl.BlockSpec((n, d0), zero),
            pl.BlockSpec((d0, d1), zero),
            pl.BlockSpec((1, d1), zero),
            pl.BlockSpec((d0, Wa1.shape[1]), zero),
            pl.BlockSpec((d1, d2), zero),
            pl.BlockSpec((1, d2), zero),
            pl.BlockSpec((d1, Wa2.shape[1]), zero),
            pl.BlockSpec(memory_space=pl.ANY),
        ],
        out_specs=pl.BlockSpec((bm, d2), out_idx),
        out_shape=jax.ShapeDtypeStruct((n, d2), jnp.float32),
        scratch_shapes=[
            pltpu.VMEM((n, d1), jnp.bfloat16),
            pltpu.VMEM((n, d1), jnp.float32),
            pltpu.VMEM((n, d1), jnp.float32),
            pltpu.VMEM((n, d2), jnp.bfloat16),
            pltpu.VMEM((n, d2), jnp.float32),
            pltpu.VMEM((_NBUF, bm, n), jnp.float32),
            pltpu.SemaphoreType.DMA((_NBUF,)),
        ],
        compiler_params=pltpu.CompilerParams(
            dimension_semantics=("arbitrary",),
        ),
    )(x, W1, b1.reshape(1, d1), Wa1, W2, b2_eff, Wa2, adj)
